# baseline (speedup 1.0000x reference)
import jax
import jax.numpy as jnp
from jax.experimental import pallas as pl
from jax.experimental.pallas import tpu as pltpu

_BM = 1024
_BK = 1024


def _body(x_ref, L_ref, xm_ref, w_ref, b_ref, out_ref, y_ref):
    n, d = x_ref.shape
    m = pl.program_id(0)
    k = pl.program_id(1)

    @pl.when((m == 0) & (k == 0))
    def _():
        y_ref[...] = jax.lax.dot_general(
            x_ref[...], w_ref[:, d:],
            (((1,), (1,)), ((), ())),
            preferred_element_type=jnp.float32)

    @pl.when(k == 0)
    def _():
        out_ref[...] = jax.lax.dot_general(
            xm_ref[...], w_ref[:, :d],
            (((1,), (1,)), ((), ())),
            preferred_element_type=jnp.float32) + b_ref[...]

    out_ref[...] += jax.lax.dot_general(
        L_ref[...], y_ref[pl.ds(k * _BK, _BK), :],
        (((1,), (0,)), ((), ())),
        preferred_element_type=jnp.float32)


def kernel(L, x, W, b):
    n, d = x.shape
    out_dim = W.shape[0]
    b2 = b.reshape(1, out_dim)
    return pl.pallas_call(
        _body,
        grid=(n // _BM, n // _BK),
        in_specs=[
            pl.BlockSpec((n, d), lambda m, k: (0, 0)),
            pl.BlockSpec((_BM, _BK), lambda m, k: (m, k)),
            pl.BlockSpec((_BM, d), lambda m, k: (m, 0)),
            pl.BlockSpec((out_dim, 2 * d), lambda m, k: (0, 0)),
            pl.BlockSpec((1, out_dim), lambda m, k: (0, 0)),
        ],
        out_specs=pl.BlockSpec((_BM, out_dim), lambda m, k: (m, 0)),
        out_shape=jax.ShapeDtypeStruct((n, out_dim), jnp.float32),
        scratch_shapes=[pltpu.VMEM((n, out_dim), jnp.float32)],
        compiler_params=pltpu.CompilerParams(
            dimension_semantics=("arbitrary", "arbitrary"),
        ),
    )(x, L, x, W, b2)


# speedup vs baseline: 1.1589x; 1.1589x over previous
import jax
import jax.numpy as jnp
from jax.experimental import pallas as pl
from jax.experimental.pallas import tpu as pltpu

_BK = 512


def _body(x_ref, L_ref, w_ref, b_ref, out_ref, y_ref):
    n, d = x_ref.shape
    k = pl.program_id(0)

    @pl.when(k == 0)
    def _():
        y_ref[...] = jax.lax.dot_general(
            x_ref[...], w_ref[:, d:],
            (((1,), (1,)), ((), ())),
            preferred_element_type=jnp.float32)

    part = jax.lax.dot_general(
        L_ref[...], y_ref[pl.ds(k * _BK, _BK), :],
        (((1,), (0,)), ((), ())),
        preferred_element_type=jnp.float32)

    @pl.when(k == 0)
    def _():
        base = jax.lax.dot_general(
            x_ref[...], w_ref[:, :d],
            (((1,), (1,)), ((), ())),
            preferred_element_type=jnp.float32)
        out_ref[...] = part + base + b_ref[...]

    @pl.when(k > 0)
    def _():
        out_ref[...] += part


def kernel(L, x, W, b):
    n, d = x.shape
    out_dim = W.shape[0]
    b2 = b.reshape(1, out_dim)
    return pl.pallas_call(
        _body,
        grid=(n // _BK,),
        in_specs=[
            pl.BlockSpec((n, d), lambda k: (0, 0)),
            pl.BlockSpec((n, _BK), lambda k: (0, k)),
            pl.BlockSpec((out_dim, 2 * d), lambda k: (0, 0)),
            pl.BlockSpec((1, out_dim), lambda k: (0, 0)),
        ],
        out_specs=pl.BlockSpec((n, out_dim), lambda k: (0, 0)),
        out_shape=jax.ShapeDtypeStruct((n, out_dim), jnp.float32),
        scratch_shapes=[pltpu.VMEM((n, out_dim), jnp.float32)],
        compiler_params=pltpu.CompilerParams(
            dimension_semantics=("arbitrary",),
        ),
    )(x, L, W, b2)


# speedup vs baseline: 1.1817x; 1.0197x over previous
import jax
import jax.numpy as jnp
from jax.experimental import pallas as pl
from jax.experimental.pallas import tpu as pltpu

_BK = 512


def _body(x_ref, L_ref, w_ref, b_ref, out_ref, y_ref):
    n, d = x_ref.shape
    k = pl.program_id(0)

    @pl.when(k == 0)
    def _():
        y_ref[...] = jax.lax.dot_general(
            x_ref[...], w_ref[:, d:],
            (((1,), (1,)), ((), ())),
            preferred_element_type=jnp.float32)

        out_ref[...] = jax.lax.dot_general(
            x_ref[...], w_ref[:, :d],
            (((1,), (1,)), ((), ())),
            preferred_element_type=jnp.float32) + b_ref[...]

    out_ref[...] += jax.lax.dot_general(
        L_ref[...], y_ref[pl.ds(k * _BK, _BK), :],
        (((1,), (0,)), ((), ())),
        preferred_element_type=jnp.float32)


def kernel(L, x, W, b):
    n, d = x.shape
    out_dim = W.shape[0]
    b2 = b.reshape(1, out_dim)
    return pl.pallas_call(
        _body,
        grid=(n // _BK,),
        in_specs=[
            pl.BlockSpec((n, d), lambda k: (0, 0)),
            pl.BlockSpec((n, _BK), lambda k: (0, k)),
            pl.BlockSpec((out_dim, 2 * d), lambda k: (0, 0)),
            pl.BlockSpec((1, out_dim), lambda k: (0, 0)),
        ],
        out_specs=pl.BlockSpec((n, out_dim), lambda k: (0, 0)),
        out_shape=jax.ShapeDtypeStruct((n, out_dim), jnp.float32),
        scratch_shapes=[pltpu.VMEM((n, out_dim), jnp.float32)],
        compiler_params=pltpu.CompilerParams(
            dimension_semantics=("arbitrary",),
        ),
    )(x, L, W, b2)
